# baseline (speedup 1.0000x reference)
import jax
import jax.numpy as jnp
from jax.experimental import pallas as pl
from jax.experimental.pallas import tpu as pltpu

_ENT_CHUNKS = 10


def _dma_copy(ent_in, rel_in, ent_out, rel_out, ent_sem, rel_sem):
    n = ent_in.shape[0]
    rows = n // _ENT_CHUNKS
    copies = []
    for i in range(_ENT_CHUNKS):
        sl = pl.ds(i * rows, rows)
        copies.append(pltpu.make_async_copy(
            ent_in.at[sl], ent_out.at[sl], ent_sem.at[i]))
    rel_copy = pltpu.make_async_copy(rel_in, rel_out, rel_sem)
    for c in copies:
        c.start()
    rel_copy.start()
    for c in copies:
        c.wait()
    rel_copy.wait()


def kernel(emb_ent, emb_rel, edge_index, rel, edge_index_all, rel_all):
    return tuple(pl.pallas_call(
        _dma_copy,
        in_specs=[pl.BlockSpec(memory_space=pl.ANY),
                  pl.BlockSpec(memory_space=pl.ANY)],
        out_specs=[pl.BlockSpec(memory_space=pl.ANY),
                   pl.BlockSpec(memory_space=pl.ANY)],
        out_shape=[jax.ShapeDtypeStruct(emb_ent.shape, emb_ent.dtype),
                   jax.ShapeDtypeStruct(emb_rel.shape, emb_rel.dtype)],
        scratch_shapes=[pltpu.SemaphoreType.DMA((_ENT_CHUNKS,)),
                        pltpu.SemaphoreType.DMA],
    )(emb_ent, emb_rel))


# speedup vs baseline: 48.1464x; 48.1464x over previous
import jax
import jax.numpy as jnp
from jax.experimental import pallas as pl
from jax.experimental.pallas import tpu as pltpu

_ENT_BLOCK = 4000


def _copy_block(src_ref, dst_ref):
    dst_ref[...] = src_ref[...]


def kernel(emb_ent, emb_rel, edge_index, rel, edge_index_all, rel_all):
    n, d = emb_ent.shape
    ent_out = pl.pallas_call(
        _copy_block,
        grid=(n // _ENT_BLOCK,),
        in_specs=[pl.BlockSpec((_ENT_BLOCK, d), lambda i: (i, 0))],
        out_specs=pl.BlockSpec((_ENT_BLOCK, d), lambda i: (i, 0)),
        out_shape=jax.ShapeDtypeStruct((n, d), emb_ent.dtype),
        compiler_params=pltpu.CompilerParams(
            dimension_semantics=("parallel",)),
    )(emb_ent)
    rel_out = pl.pallas_call(
        _copy_block,
        out_shape=jax.ShapeDtypeStruct(emb_rel.shape, emb_rel.dtype),
    )(emb_rel)
    return (ent_out, rel_out)


# speedup vs baseline: 48.5665x; 1.0087x over previous
import jax
import jax.numpy as jnp
from jax.experimental import pallas as pl
from jax.experimental.pallas import tpu as pltpu

_ENT_BLOCK = 4000


def _copy_both(ent_ref, rel_ref, ent_out, rel_out):
    ent_out[...] = ent_ref[...]

    @pl.when(pl.program_id(0) == 0)
    def _():
        rel_out[...] = rel_ref[...]


def kernel(emb_ent, emb_rel, edge_index, rel, edge_index_all, rel_all):
    n, d = emb_ent.shape
    m, r = emb_rel.shape
    ent_out, rel_out = pl.pallas_call(
        _copy_both,
        grid=(n // _ENT_BLOCK,),
        in_specs=[pl.BlockSpec((_ENT_BLOCK, d), lambda i: (i, 0)),
                  pl.BlockSpec((m, r), lambda i: (0, 0))],
        out_specs=[pl.BlockSpec((_ENT_BLOCK, d), lambda i: (i, 0)),
                   pl.BlockSpec((m, r), lambda i: (0, 0))],
        out_shape=[jax.ShapeDtypeStruct((n, d), emb_ent.dtype),
                   jax.ShapeDtypeStruct((m, r), emb_rel.dtype)],
        compiler_params=pltpu.CompilerParams(
            dimension_semantics=("arbitrary",)),
    )(emb_ent, emb_rel)
    return (ent_out, rel_out)
